# baseline (speedup 1.0000x reference)
import functools

import jax
import jax.numpy as jnp
from jax import lax
from jax.experimental import pallas as pl
from jax.experimental.pallas import tpu as pltpu
from jax.experimental.pallas import tpu_sc as plsc

BATCH = 16384
EMB_DIM = 4096
NUM_CORES = 2
NUM_SUBCORES = 16
NUM_WORKERS = NUM_CORES * NUM_SUBCORES
BPW = BATCH // NUM_WORKERS
K = 8
CHW = K * EMB_DIM
UNROLL = 8
ADD_ITERS = EMB_DIM // 16 // UNROLL


def kernel(condition, image_emb, emb_table):
    mesh = plsc.VectorSubcoreMesh(core_axis_name="c", subcore_axis_name="s")

    @functools.partial(
        pl.kernel,
        mesh=mesh,
        out_type=jax.ShapeDtypeStruct((BATCH, EMB_DIM), jnp.float32),
        scratch_types=[
            pltpu.VMEM((BPW,), jnp.int32),
            pltpu.VMEM((K, EMB_DIM), jnp.float32),
            pltpu.VMEM((K, EMB_DIM), jnp.float32),
            pltpu.VMEM((K, EMB_DIM), jnp.float32),
            pltpu.SemaphoreType.DMA,
            pltpu.SemaphoreType.DMA,
            pltpu.SemaphoreType.DMA,
            pltpu.SemaphoreType.DMA,
        ],
    )
    def run(cond_hbm, img_hbm, table_hbm, out_hbm,
            idx_v, rows0, rows1, img_v, sem_g, sem_i, sem_o0, sem_o1):
        wid = lax.axis_index("s") * NUM_CORES + lax.axis_index("c")
        base = wid * BPW
        pltpu.sync_copy(cond_hbm.at[pl.ds(base, BPW)], idx_v)

        rows_bufs = (rows0, rows1)
        out_sems = (sem_o0, sem_o1)

        def drain(sem, buf):
            pltpu.make_async_copy(table_hbm.at[pl.ds(0, K)], buf, sem).wait()

        def group_body(g, carry):
            for half in range(2):
                j = g * 2 + half
                rows_b = rows_bufs[half]
                sem_o = out_sems[half]
                start = base + j * K

                im = pltpu.async_copy(
                    img_hbm.at[pl.ds(start, K)], img_v, sem_i
                )
                @pl.when(g > 0)
                def _(rows_b=rows_b, sem_o=sem_o):
                    drain(sem_o, rows_b)

                gth = pltpu.async_copy(
                    table_hbm.at[idx_v.at[pl.ds(j * K, K)]], rows_b, sem_g
                )
                gth.wait()
                im.wait()

                for r in range(K):
                    def add_body(t, c2, r=r, rows_b=rows_b):
                        for uu in range(UNROLL):
                            sl = pl.ds((t * UNROLL + uu) * 16, 16)
                            rows_b[r, sl] = rows_b[r, sl] + img_v[r, sl]
                        return c2

                    lax.fori_loop(0, ADD_ITERS, add_body, 0)

                pltpu.async_copy(rows_b, out_hbm.at[pl.ds(start, K)], sem_o)
            return carry

        lax.fori_loop(0, BPW // 16, group_body, 0)
        drain(sem_o0, rows0)
        drain(sem_o1, rows1)

    return run(condition, image_emb, emb_table)


# speedup vs baseline: 1.0045x; 1.0045x over previous
import functools

import jax
import jax.numpy as jnp
from jax import lax
from jax.experimental import pallas as pl
from jax.experimental.pallas import tpu as pltpu
from jax.experimental.pallas import tpu_sc as plsc

BATCH = 16384
EMB_DIM = 4096
NUM_CORES = 2
NUM_SUBCORES = 16
NUM_WORKERS = NUM_CORES * NUM_SUBCORES
BPW = BATCH // NUM_WORKERS
K = 8
CHW = K * EMB_DIM
UNROLL = 8
ADD_ITERS = EMB_DIM // 16 // UNROLL


def kernel(condition, image_emb, emb_table):
    mesh = plsc.VectorSubcoreMesh(core_axis_name="c", subcore_axis_name="s")

    @functools.partial(
        pl.kernel,
        mesh=mesh,
        out_type=jax.ShapeDtypeStruct((BATCH, EMB_DIM), jnp.float32),
        scratch_types=[
            pltpu.VMEM((BPW,), jnp.int32),
            pltpu.VMEM((K, EMB_DIM), jnp.float32),
            pltpu.VMEM((K, EMB_DIM), jnp.float32),
            pltpu.VMEM((K, EMB_DIM), jnp.float32),
            pltpu.SemaphoreType.DMA,
            pltpu.SemaphoreType.DMA,
            pltpu.SemaphoreType.DMA,
            pltpu.SemaphoreType.DMA,
        ],
    )
    def run(cond_hbm, img_hbm, table_hbm, out_hbm,
            idx_v, rows0, rows1, img_v, sem_g, sem_i, sem_o0, sem_o1):
        wid = lax.axis_index("s") * NUM_CORES + lax.axis_index("c")
        base = wid * BPW
        pltpu.sync_copy(cond_hbm.at[pl.ds(base, BPW)], idx_v)

        rows_bufs = (rows0, rows1)
        out_sems = (sem_o0, sem_o1)

        def drain(sem, buf):
            pltpu.make_async_copy(table_hbm.at[pl.ds(0, K)], buf, sem).wait()

        def group_body(g, carry):
            for half in range(2):
                j = g * 2 + half
                rows_b = rows_bufs[half]
                sem_o = out_sems[half]
                start = base + j * K

                @pl.when(g > 0)
                def _(rows_b=rows_b, sem_o=sem_o):
                    drain(sem_o, rows_b)

                gth = pltpu.async_copy(
                    table_hbm.at[idx_v.at[pl.ds(j * K, K)]], rows_b, sem_g
                )
                im = pltpu.async_copy(
                    img_hbm.at[pl.ds(start, K)], img_v, sem_i
                )
                gth.wait()
                im.wait()

                for r in range(K):
                    def add_body(t, c2, r=r, rows_b=rows_b):
                        for uu in range(UNROLL):
                            sl = pl.ds((t * UNROLL + uu) * 16, 16)
                            rows_b[r, sl] = rows_b[r, sl] + img_v[r, sl]
                        return c2

                    lax.fori_loop(0, ADD_ITERS, add_body, 0)

                pltpu.async_copy(rows_b, out_hbm.at[pl.ds(start, K)], sem_o)
            return carry

        lax.fori_loop(0, BPW // 16, group_body, 0)
        drain(sem_o0, rows0)
        drain(sem_o1, rows1)

    return run(condition, image_emb, emb_table)
